# baseline (speedup 1.0000x reference)
import functools

import jax
import jax.numpy as jnp
from jax import lax
from jax.experimental import pallas as pl
from jax.experimental.pallas import tpu as pltpu
from jax.experimental.pallas import tpu_sc as plsc

VOCAB = 100000
EMBED_DIM = 128
B_ROWS = 4096
B_COLS = 50

NC = 2
NS = 16
NW = NC * NS

N_SPLIT = 2
ROWS_SPLIT = B_ROWS // N_SPLIT
MAJ_BLK = 8


def _make_sc_gather(n_rows):
  b = n_rows * B_COLS
  b_per_w = b // NW
  chunk = 400
  n_chunks = b_per_w // chunk

  mesh = plsc.VectorSubcoreMesh(
      core_axis_name="c", subcore_axis_name="s",
      num_cores=NC, num_subcores=NS)

  @functools.partial(
      pl.kernel,
      out_type=jax.ShapeDtypeStruct((b, EMBED_DIM), jnp.float32),
      mesh=mesh,
      scratch_types=[
          pltpu.VMEM((b_per_w,), jnp.int32),
          pltpu.VMEM((chunk, EMBED_DIM), jnp.float32),
          pltpu.VMEM((chunk, EMBED_DIM), jnp.float32),
          pltpu.SemaphoreType.DMA,
          pltpu.SemaphoreType.DMA,
          pltpu.SemaphoreType.DMA,
          pltpu.SemaphoreType.DMA,
      ],
  )
  def gather_kernel(idx_hbm, table_hbm, out_hbm, idx_v, rows0, rows1,
                    g0, g1, s0, s1):
    wid = lax.axis_index("s") * NC + lax.axis_index("c")
    base = wid * b_per_w
    pltpu.sync_copy(idx_hbm.at[pl.ds(base, b_per_w)], idx_v)

    def gather(c, buf, sem):
      return pltpu.make_async_copy(
          table_hbm.at[idx_v.at[pl.ds(c * chunk, chunk)]], buf, sem)

    def put(c, buf, sem):
      return pltpu.make_async_copy(
          buf, out_hbm.at[pl.ds(base + c * chunk, chunk)], sem)

    gather(0, rows0, g0).start()
    gather(1, rows1, g1).start()

    def body(i, carry):
      c = i * 2
      gather(c, rows0, g0).wait()
      put(c, rows0, s0).start()
      gather(c + 1, rows1, g1).wait()
      put(c + 1, rows1, s1).start()
      put(c, rows0, s0).wait()

      @pl.when(c + 2 < n_chunks)
      def _():
        gather(c + 2, rows0, g0).start()

      put(c + 1, rows1, s1).wait()

      @pl.when(c + 3 < n_chunks)
      def _():
        gather(c + 3, rows1, g1).start()

      return carry

    lax.fori_loop(0, n_chunks // 2, body, 0)

  return gather_kernel


_SC_GATHER = _make_sc_gather(ROWS_SPLIT)


def _relayout_body(flat_ref, out_ref):
  for m in range(MAJ_BLK):
    out_ref[m] = flat_ref[pl.ds(m * B_COLS, B_COLS), :]


def _relayout_first(flat):
  return pl.pallas_call(
      _relayout_body,
      grid=(ROWS_SPLIT // MAJ_BLK,),
      in_specs=[pl.BlockSpec((MAJ_BLK * B_COLS, EMBED_DIM),
                             lambda g: (g, 0))],
      out_specs=pl.BlockSpec((MAJ_BLK, B_COLS, EMBED_DIM),
                             lambda g: (g, 0, 0)),
      out_shape=jax.ShapeDtypeStruct((B_ROWS, B_COLS, EMBED_DIM),
                                     jnp.float32),
  )(flat)


def _relayout_second(flat, partial):
  def body(flat_ref, _partial_ref, out_ref):
    _relayout_body(flat_ref, out_ref)

  off = ROWS_SPLIT // MAJ_BLK
  return pl.pallas_call(
      body,
      grid=(ROWS_SPLIT // MAJ_BLK,),
      in_specs=[
          pl.BlockSpec((MAJ_BLK * B_COLS, EMBED_DIM), lambda g: (g, 0)),
          pl.BlockSpec(memory_space=pl.ANY),
      ],
      out_specs=pl.BlockSpec((MAJ_BLK, B_COLS, EMBED_DIM),
                             lambda g: (g + off, 0, 0)),
      out_shape=jax.ShapeDtypeStruct((B_ROWS, B_COLS, EMBED_DIM),
                                     jnp.float32),
      input_output_aliases={1: 0},
  )(flat, partial)


def kernel(x, table):
  idx = x.astype(jnp.int32)
  flats = [
      _SC_GATHER(idx[i * ROWS_SPLIT:(i + 1) * ROWS_SPLIT].reshape(-1), table)
      for i in range(N_SPLIT)
  ]
  partial = _relayout_first(flats[0])
  return _relayout_second(flats[1], partial)


# speedup vs baseline: 2.5915x; 2.5915x over previous
import functools

import jax
import jax.numpy as jnp
from jax import lax
from jax.experimental import pallas as pl
from jax.experimental.pallas import tpu as pltpu
from jax.experimental.pallas import tpu_sc as plsc

VOCAB = 100000
EMBED_DIM = 128
B_ROWS = 4096
B_COLS = 50
B = B_ROWS * B_COLS

NC = 2
NS = 16
NW = NC * NS
B_PER_W = B // NW
MAJ_PER_W = B_ROWS // NW
CHUNK = 400
MAJ_PER_CHUNK = CHUNK // B_COLS
N_CHUNKS = B_PER_W // CHUNK


def _make_kernel():
  mesh = plsc.VectorSubcoreMesh(
      core_axis_name="c", subcore_axis_name="s",
      num_cores=NC, num_subcores=NS)

  @functools.partial(
      pl.kernel,
      out_type=jax.ShapeDtypeStruct((B_ROWS, B_COLS, EMBED_DIM), jnp.float32),
      mesh=mesh,
      scratch_types=[
          pltpu.VMEM((B_PER_W,), jnp.int32),
          pltpu.VMEM((CHUNK, EMBED_DIM), jnp.float32),
          pltpu.VMEM((CHUNK, EMBED_DIM), jnp.float32),
          pltpu.SemaphoreType.DMA,
          pltpu.SemaphoreType.DMA,
          pltpu.SemaphoreType.DMA,
          pltpu.SemaphoreType.DMA,
      ],
  )
  def gather_kernel(idx_hbm, table_hbm, out_hbm, idx_v, rows0, rows1,
                    g0, g1, s0, s1):
    wid = lax.axis_index("s") * NC + lax.axis_index("c")
    base = wid * B_PER_W
    maj0 = wid * MAJ_PER_W
    pltpu.sync_copy(idx_hbm.at[pl.ds(base, B_PER_W)], idx_v)

    def gather(c, buf, sem):
      return pltpu.make_async_copy(
          table_hbm.at[idx_v.at[pl.ds(c * CHUNK, CHUNK)]], buf, sem)

    def put_start(c, buf, sem):
      m0 = maj0 + c * MAJ_PER_CHUNK
      for j in range(MAJ_PER_CHUNK):
        pltpu.make_async_copy(
            buf.at[pl.ds(j * B_COLS, B_COLS)], out_hbm.at[m0 + j], sem
        ).start()

    def put_wait(c, buf, sem):
      m0 = maj0 + c * MAJ_PER_CHUNK
      for j in range(MAJ_PER_CHUNK):
        pltpu.make_async_copy(
            buf.at[pl.ds(j * B_COLS, B_COLS)], out_hbm.at[m0 + j], sem
        ).wait()

    gather(0, rows0, g0).start()
    gather(1, rows1, g1).start()

    def body(i, carry):
      c = i * 2
      gather(c, rows0, g0).wait()
      put_start(c, rows0, s0)
      gather(c + 1, rows1, g1).wait()
      put_start(c + 1, rows1, s1)
      put_wait(c, rows0, s0)

      @pl.when(c + 2 < N_CHUNKS)
      def _():
        gather(c + 2, rows0, g0).start()

      put_wait(c + 1, rows1, s1)

      @pl.when(c + 3 < N_CHUNKS)
      def _():
        gather(c + 3, rows1, g1).start()

      return carry

    lax.fori_loop(0, N_CHUNKS // 2, body, 0)

  return gather_kernel


_GATHER = _make_kernel()


def kernel(x, table):
  idx = x.reshape(-1).astype(jnp.int32)
  return _GATHER(idx, table)
